# baseline (speedup 1.0000x reference)
import functools

import jax
import jax.numpy as jnp
from jax import lax
from jax.experimental import pallas as pl
from jax.experimental.pallas import tpu as pltpu
from jax.experimental.pallas import tpu_sc as plsc

N_TOK = 8192
N_EXP = 64
K = 8
BLOCK = 1024

NUM_WORKERS = 32
RPW = N_TOK // NUM_WORKERS
L = 16
NG = N_EXP // L


def _probs_kernel(x_ref, w_ref, p_ref):
    x = x_ref[...]
    w = w_ref[...]
    logits = jax.lax.dot_general(
        x, w, (((1,), (1,)), ((), ())), preferred_element_type=jnp.float32
    )
    m = jnp.max(logits, axis=1, keepdims=True)
    e = jnp.exp(logits - m)
    s = jnp.sum(e, axis=1, keepdims=True)
    p_ref[...] = jnp.concatenate(
        [e / s, jnp.full((BLOCK, 128 - N_EXP), -1.0, jnp.float32)], axis=1
    )


def _tc_probs(hidden_states, weight):
    return pl.pallas_call(
        _probs_kernel,
        grid=(N_TOK // BLOCK,),
        in_specs=[
            pl.BlockSpec((BLOCK, N_EXP), lambda i: (i, 0)),
            pl.BlockSpec((N_EXP, N_EXP), lambda i: (0, 0)),
        ],
        out_specs=pl.BlockSpec((BLOCK, 128), lambda i: (i, 0)),
        out_shape=jax.ShapeDtypeStruct((N_TOK, 128), jnp.float32),
    )(hidden_states, weight)


def _slice_kernel(w_ref, i_ref, ow_ref, oi_ref):
    ow_ref[...] = w_ref[:, :K]
    oi_ref[...] = i_ref[:, :K]


def _tc_slice(w16, i16):
    return pl.pallas_call(
        _slice_kernel,
        grid=(N_TOK // BLOCK,),
        in_specs=[
            pl.BlockSpec((BLOCK, L), lambda i: (i, 0)),
            pl.BlockSpec((BLOCK, L), lambda i: (i, 0)),
        ],
        out_specs=[
            pl.BlockSpec((BLOCK, K), lambda i: (i, 0)),
            pl.BlockSpec((BLOCK, K), lambda i: (i, 0)),
        ],
        out_shape=[
            jax.ShapeDtypeStruct((N_TOK, K), jnp.float32),
            jax.ShapeDtypeStruct((N_TOK, K), jnp.int32),
        ],
    )(w16, i16)


@functools.partial(
    pl.kernel,
    out_type=[
        jax.ShapeDtypeStruct((N_TOK, L), jnp.float32),
        jax.ShapeDtypeStruct((N_TOK, L), jnp.int32),
    ],
    mesh=plsc.VectorSubcoreMesh(core_axis_name="c", subcore_axis_name="s"),
    compiler_params=pltpu.CompilerParams(needs_layout_passes=False),
    scratch_types=[
        pltpu.VMEM((RPW, 128), jnp.float32),
        pltpu.VMEM((RPW, L), jnp.float32),
        pltpu.VMEM((RPW, L), jnp.int32),
    ],
)
def _topk_sc(p_hbm, out_w_hbm, out_i_hbm, buf, ow, oi):
    wid = lax.axis_index("s") * 2 + lax.axis_index("c")
    base = pl.multiple_of(wid * RPW, RPW)
    pltpu.sync_copy(p_hbm.at[pl.ds(base, RPW)], buf)

    lane = lax.iota(jnp.int32, L)
    iotas = [lane + g * L for g in range(NG)]
    lane_next = jnp.minimum(lane + 1, L - 1)
    lane_prev = jnp.maximum(lane - 1, 0)
    lane_m8 = jnp.maximum(lane - K, 0)
    is_last = lane == (L - 1)
    is_first = lane == 0
    lo_half = lane < K

    def _gath(x, i):
        return x.at[i].get(mode="promise_in_bounds")

    def _merge(ak, ai, bk, bi):
        rbk = jnp.flip(bk, 0)
        rbi = jnp.flip(bi, 0)
        take = ak >= rbk
        mk = jnp.where(take, ak, rbk)
        mi = jnp.where(take, ai, rbi)
        return plsc.sort_key_val(mk, mi, descending=True)

    def topk_one_row(r):
        sk, si = [], []
        for g in range(NG):
            k_g, i_g = plsc.sort_key_val(
                buf[r, pl.ds(g * L, L)],
                iotas[g],
                descending=True,
            )
            sk.append(k_g)
            si.append(i_g)
        k01, i01 = _merge(sk[0], si[0], sk[1], si[1])
        k23, i23 = _merge(sk[2], si[2], sk[3], si[3])
        kf, idxf = _merge(k01, i01, k23, i23)
        kn = jnp.where(is_last, -1.0, _gath(kf, lane_next))
        inx = _gath(idxf, lane_next)
        kp = jnp.where(is_first, -2.0, _gath(kf, lane_prev))
        ipv = _gath(idxf, lane_prev)
        fixed = jnp.where(kf == kn, jnp.minimum(idxf, inx), idxf)
        fixed = jnp.where(kf == kp, jnp.maximum(fixed, ipv), fixed)
        return kf, fixed

    def pair_body(p, carry):
        w_a, i_a = topk_one_row(p * 2)
        w_b, i_b = topk_one_row(p * 2 + 1)
        ow[p * 2, :] = w_a
        oi[p * 2, :] = i_a
        ow[p * 2 + 1, :] = w_b
        oi[p * 2 + 1, :] = i_b
        return carry

    lax.fori_loop(0, RPW // 2, pair_body, 0)

    pltpu.sync_copy(ow, out_w_hbm.at[pl.ds(base, RPW)])
    pltpu.sync_copy(oi, out_i_hbm.at[pl.ds(base, RPW)])


@jax.jit
def kernel(hidden_states, weight):
    probs = _tc_probs(hidden_states, weight)
    w16, i16 = _topk_sc(probs)
    return _tc_slice(w16, i16)


# speedup vs baseline: 1.0769x; 1.0769x over previous
import functools

import jax
import jax.numpy as jnp
from jax import lax
from jax.experimental import pallas as pl
from jax.experimental.pallas import tpu as pltpu
from jax.experimental.pallas import tpu_sc as plsc

N_TOK = 8192
N_EXP = 64
K = 8
BLOCK = 1024

NUM_WORKERS = 32
RPW = N_TOK // NUM_WORKERS
L = 16
NG = N_EXP // L


def _probs_kernel(x_ref, w_ref, p_ref):
    x = x_ref[...]
    w = w_ref[...]
    logits = jax.lax.dot_general(
        x, w, (((1,), (1,)), ((), ())), preferred_element_type=jnp.float32
    )
    m = jnp.max(logits, axis=1, keepdims=True)
    e = jnp.exp(logits - m)
    s = jnp.sum(e, axis=1, keepdims=True)
    p_ref[...] = jnp.concatenate(
        [e / s, jnp.full((BLOCK, 128 - N_EXP), -1.0, jnp.float32)], axis=1
    )


def _tc_probs(hidden_states, weight):
    return pl.pallas_call(
        _probs_kernel,
        grid=(N_TOK // BLOCK,),
        in_specs=[
            pl.BlockSpec((BLOCK, N_EXP), lambda i: (i, 0)),
            pl.BlockSpec((N_EXP, N_EXP), lambda i: (0, 0)),
        ],
        out_specs=pl.BlockSpec((BLOCK, 128), lambda i: (i, 0)),
        out_shape=jax.ShapeDtypeStruct((N_TOK, 128), jnp.float32),
    )(hidden_states, weight)


def _slice_kernel(w_ref, i_ref, ow_ref, oi_ref):
    ow_ref[...] = w_ref[:, :K]
    oi_ref[...] = i_ref[:, :K]


def _tc_slice(w16, i16):
    return pl.pallas_call(
        _slice_kernel,
        grid=(N_TOK // BLOCK,),
        in_specs=[
            pl.BlockSpec((BLOCK, L), lambda i: (i, 0)),
            pl.BlockSpec((BLOCK, L), lambda i: (i, 0)),
        ],
        out_specs=[
            pl.BlockSpec((BLOCK, K), lambda i: (i, 0)),
            pl.BlockSpec((BLOCK, K), lambda i: (i, 0)),
        ],
        out_shape=[
            jax.ShapeDtypeStruct((N_TOK, K), jnp.float32),
            jax.ShapeDtypeStruct((N_TOK, K), jnp.int32),
        ],
    )(w16, i16)


@functools.partial(
    pl.kernel,
    out_type=[
        jax.ShapeDtypeStruct((N_TOK * K,), jnp.float32),
        jax.ShapeDtypeStruct((N_TOK * K,), jnp.int32),
    ],
    mesh=plsc.VectorSubcoreMesh(core_axis_name="c", subcore_axis_name="s"),
    compiler_params=pltpu.CompilerParams(needs_layout_passes=False),
    scratch_types=[
        pltpu.VMEM((RPW, 128), jnp.float32),
        pltpu.VMEM((RPW * K + K,), jnp.float32),
        pltpu.VMEM((RPW * K + K,), jnp.int32),
    ],
)
def _topk_sc(p_hbm, out_w_hbm, out_i_hbm, buf, ow, oi):
    wid = lax.axis_index("s") * 2 + lax.axis_index("c")
    base = pl.multiple_of(wid * RPW, RPW)
    pltpu.sync_copy(p_hbm.at[pl.ds(base, RPW)], buf)

    lane = lax.iota(jnp.int32, L)
    iotas = [lane + g * L for g in range(NG)]
    lane_next = jnp.minimum(lane + 1, L - 1)
    lane_prev = jnp.maximum(lane - 1, 0)
    lane_m8 = jnp.maximum(lane - K, 0)
    is_last = lane == (L - 1)
    is_first = lane == 0
    lo_half = lane < K

    def _gath(x, i):
        return x.at[i].get(mode="promise_in_bounds")

    def _merge(ak, ai, bk, bi):
        rbk = jnp.flip(bk, 0)
        rbi = jnp.flip(bi, 0)
        take = ak >= rbk
        mk = jnp.where(take, ak, rbk)
        mi = jnp.where(take, ai, rbi)
        return plsc.sort_key_val(mk, mi, descending=True)

    def topk_one_row(r):
        sk, si = [], []
        for g in range(NG):
            k_g, i_g = plsc.sort_key_val(
                buf[r, pl.ds(g * L, L)],
                iotas[g],
                descending=True,
            )
            sk.append(k_g)
            si.append(i_g)
        k01, i01 = _merge(sk[0], si[0], sk[1], si[1])
        k23, i23 = _merge(sk[2], si[2], sk[3], si[3])
        kf, idxf = _merge(k01, i01, k23, i23)
        kn = jnp.where(is_last, -1.0, _gath(kf, lane_next))
        inx = _gath(idxf, lane_next)
        kp = jnp.where(is_first, -2.0, _gath(kf, lane_prev))
        ipv = _gath(idxf, lane_prev)
        fixed = jnp.where(kf == kn, jnp.minimum(idxf, inx), idxf)
        fixed = jnp.where(kf == kp, jnp.maximum(fixed, ipv), fixed)
        return kf, fixed

    def pair_body(p, carry):
        w_a, i_a = topk_one_row(p * 2)
        w_b, i_b = topk_one_row(p * 2 + 1)
        plsc.store_compressed(ow.at[pl.ds(p * 2 * K, L)], w_a, mask=lo_half)
        plsc.store_compressed(oi.at[pl.ds(p * 2 * K, L)], i_a, mask=lo_half)
        plsc.store_compressed(ow.at[pl.ds((p * 2 + 1) * K, L)], w_b, mask=lo_half)
        plsc.store_compressed(oi.at[pl.ds((p * 2 + 1) * K, L)], i_b, mask=lo_half)
        return carry

    lax.fori_loop(0, RPW // 2, pair_body, 0)

    pltpu.sync_copy(
        ow.at[pl.ds(0, RPW * K)], out_w_hbm.at[pl.ds(base * K, RPW * K)]
    )
    pltpu.sync_copy(
        oi.at[pl.ds(0, RPW * K)], out_i_hbm.at[pl.ds(base * K, RPW * K)]
    )


@jax.jit
def kernel(hidden_states, weight):
    probs = _tc_probs(hidden_states, weight)
    w_flat, i_flat = _topk_sc(probs)
    return (w_flat.reshape(N_TOK, K), i_flat.reshape(N_TOK, K))


# speedup vs baseline: 1.1756x; 1.0917x over previous
import functools

import jax
import jax.numpy as jnp
from jax import lax
from jax.experimental import pallas as pl
from jax.experimental.pallas import tpu as pltpu
from jax.experimental.pallas import tpu_sc as plsc

N_TOK = 8192
N_EXP = 64
K = 8
N_CHUNKS = 2
CTOK = N_TOK // N_CHUNKS
BLOCK = 1024

NUM_WORKERS = 32
RPW = CTOK // NUM_WORKERS
L = 16
NG = N_EXP // L


def _probs_kernel(x_ref, w_ref, p_ref):
    x = x_ref[...]
    w = w_ref[...]
    logits = jax.lax.dot_general(
        x, w, (((1,), (1,)), ((), ())), preferred_element_type=jnp.float32
    )
    m = jnp.max(logits, axis=1, keepdims=True)
    e = jnp.exp(logits - m)
    s = jnp.sum(e, axis=1, keepdims=True)
    p_ref[...] = e / s


def _tc_probs(x_chunk, weight):
    return pl.pallas_call(
        _probs_kernel,
        grid=(CTOK // BLOCK,),
        in_specs=[
            pl.BlockSpec((BLOCK, N_EXP), lambda i: (i, 0)),
            pl.BlockSpec((N_EXP, N_EXP), lambda i: (0, 0)),
        ],
        out_specs=pl.BlockSpec((BLOCK, N_EXP), lambda i: (i, 0)),
        out_shape=jax.ShapeDtypeStruct((CTOK, N_EXP), jnp.float32),
    )(x_chunk, weight)


@functools.partial(
    pl.kernel,
    out_type=[
        jax.ShapeDtypeStruct((CTOK, L), jnp.float32),
        jax.ShapeDtypeStruct((CTOK, L), jnp.int32),
    ],
    mesh=plsc.VectorSubcoreMesh(core_axis_name="c", subcore_axis_name="s"),
    compiler_params=pltpu.CompilerParams(needs_layout_passes=False),
    scratch_types=[
        pltpu.VMEM((RPW, N_EXP), jnp.float32),
        pltpu.VMEM((RPW, L), jnp.float32),
        pltpu.VMEM((RPW, L), jnp.int32),
    ],
)
def _topk_sc(p_hbm, out_w_hbm, out_i_hbm, buf, ow, oi):
    wid = lax.axis_index("s") * 2 + lax.axis_index("c")
    base = pl.multiple_of(wid * RPW, RPW)
    pltpu.sync_copy(p_hbm.at[pl.ds(base, RPW)], buf)

    lane = lax.iota(jnp.int32, L)
    iotas = [lane + g * L for g in range(NG)]
    lane_next = jnp.minimum(lane + 1, L - 1)
    lane_prev = jnp.maximum(lane - 1, 0)
    is_last = lane == (L - 1)
    is_first = lane == 0

    def _gath(x, i):
        return x.at[i].get(mode="promise_in_bounds")

    def _merge(ak, ai, bk, bi):
        rbk = jnp.flip(bk, 0)
        rbi = jnp.flip(bi, 0)
        take = ak >= rbk
        mk = jnp.where(take, ak, rbk)
        mi = jnp.where(take, ai, rbi)
        return plsc.sort_key_val(mk, mi, descending=True)

    def topk_one_row(r):
        sk, si = [], []
        for g in range(NG):
            k_g, i_g = plsc.sort_key_val(
                buf[r, pl.ds(g * L, L)],
                iotas[g],
                descending=True,
            )
            sk.append(k_g)
            si.append(i_g)
        k01, i01 = _merge(sk[0], si[0], sk[1], si[1])
        k23, i23 = _merge(sk[2], si[2], sk[3], si[3])
        kf, idxf = _merge(k01, i01, k23, i23)
        kn = jnp.where(is_last, -1.0, _gath(kf, lane_next))
        inx = _gath(idxf, lane_next)
        kp = jnp.where(is_first, -2.0, _gath(kf, lane_prev))
        ipv = _gath(idxf, lane_prev)
        fixed = jnp.where(kf == kn, jnp.minimum(idxf, inx), idxf)
        fixed = jnp.where(kf == kp, jnp.maximum(fixed, ipv), fixed)
        return kf, fixed

    def pair_body(p, carry):
        w_a, i_a = topk_one_row(p * 2)
        w_b, i_b = topk_one_row(p * 2 + 1)
        ow[p * 2, :] = w_a
        oi[p * 2, :] = i_a
        ow[p * 2 + 1, :] = w_b
        oi[p * 2 + 1, :] = i_b
        return carry

    lax.fori_loop(0, RPW // 2, pair_body, 0)

    pltpu.sync_copy(ow, out_w_hbm.at[pl.ds(base, RPW)])
    pltpu.sync_copy(oi, out_i_hbm.at[pl.ds(base, RPW)])


@jax.jit
def kernel(hidden_states, weight):
    ws, idxs = [], []
    for c in range(N_CHUNKS):
        probs = _tc_probs(
            lax.slice_in_dim(hidden_states, c * CTOK, (c + 1) * CTOK), weight
        )
        w16, i16 = _topk_sc(probs)
        ws.append(w16)
        idxs.append(i16)
    out_w = jnp.concatenate(ws, axis=0)[:, :K]
    out_i = jnp.concatenate(idxs, axis=0)[:, :K]
    return (out_w, out_i)


# speedup vs baseline: 2.3012x; 1.9575x over previous
import jax
import jax.numpy as jnp
from jax.experimental import pallas as pl

N_TOK = 8192
N_EXP = 64
K = 8
BLOCK = 1024
NEG_INF = float("-inf")


def _gate_kernel(x_ref, w_ref, out_w_ref, out_i_ref):
    x = x_ref[...]
    w = w_ref[...]
    lt = jax.lax.dot_general(
        w, x, (((1,), (1,)), ((), ())), preferred_element_type=jnp.float32
    )
    eidx = jax.lax.broadcasted_iota(jnp.int32, lt.shape, 0)
    kiota = jax.lax.broadcasted_iota(jnp.int32, (K, BLOCK), 0)

    l = lt
    vals = jnp.zeros((K, BLOCK), jnp.float32)
    idxs = jnp.zeros((K, BLOCK), jnp.int32)
    m = None
    for k in range(K):
        cur = jnp.max(l, axis=0, keepdims=True)
        idx = jnp.min(jnp.where(l == cur, eidx, N_EXP), axis=0, keepdims=True)
        vals = jnp.where(kiota == k, cur, vals)
        idxs = jnp.where(kiota == k, idx, idxs)
        if k == 0:
            m = cur
            s = jnp.sum(jnp.exp(lt - m), axis=0, keepdims=True)
        if k + 1 < K:
            l = jnp.where(eidx == idx, NEG_INF, l)

    wts = jnp.exp(vals - m) / s
    out_w_ref[...] = wts.T
    out_i_ref[...] = idxs.T


@jax.jit
def kernel(hidden_states, weight):
    grid = (N_TOK // BLOCK,)
    out_w, out_i = pl.pallas_call(
        _gate_kernel,
        grid=grid,
        in_specs=[
            pl.BlockSpec((BLOCK, N_EXP), lambda i: (i, 0)),
            pl.BlockSpec((N_EXP, N_EXP), lambda i: (0, 0)),
        ],
        out_specs=[
            pl.BlockSpec((BLOCK, K), lambda i: (i, 0)),
            pl.BlockSpec((BLOCK, K), lambda i: (i, 0)),
        ],
        out_shape=[
            jax.ShapeDtypeStruct((N_TOK, K), jnp.float32),
            jax.ShapeDtypeStruct((N_TOK, K), jnp.int32),
        ],
    )(hidden_states, weight)
    return (out_w, out_i)


# speedup vs baseline: 2.4889x; 1.0816x over previous
import jax
import jax.numpy as jnp
from jax.experimental import pallas as pl

N_TOK = 8192
N_EXP = 64
K = 8
BLOCK = 2048
NEG_INF = float("-inf")


def _gate_kernel(x_ref, w_ref, out_w_ref, out_i_ref):
    x = x_ref[...]
    w = w_ref[...]
    lt = jax.lax.dot_general(
        w, x, (((1,), (1,)), ((), ())), preferred_element_type=jnp.float32
    )
    eidx = jax.lax.broadcasted_iota(jnp.int32, lt.shape, 0)
    kiota = jax.lax.broadcasted_iota(jnp.int32, (K, BLOCK), 0)

    l = lt
    vals = jnp.zeros((K, BLOCK), jnp.float32)
    idxs = jnp.zeros((K, BLOCK), jnp.int32)
    m = None
    for k in range(K):
        cur = jnp.max(l, axis=0, keepdims=True)
        idx = jnp.min(jnp.where(l == cur, eidx, N_EXP), axis=0, keepdims=True)
        vals = jnp.where(kiota == k, cur, vals)
        idxs = jnp.where(kiota == k, idx, idxs)
        if k == 0:
            m = cur
            s = jnp.sum(jnp.exp(lt - m), axis=0, keepdims=True)
        if k + 1 < K:
            l = jnp.where(eidx == idx, NEG_INF, l)

    wts = jnp.exp(vals - m) / s
    out_w_ref[...] = wts.T
    out_i_ref[...] = idxs.T


@jax.jit
def kernel(hidden_states, weight):
    grid = (N_TOK // BLOCK,)
    out_w, out_i = pl.pallas_call(
        _gate_kernel,
        grid=grid,
        in_specs=[
            pl.BlockSpec((BLOCK, N_EXP), lambda i: (i, 0)),
            pl.BlockSpec((N_EXP, N_EXP), lambda i: (0, 0)),
        ],
        out_specs=[
            pl.BlockSpec((BLOCK, K), lambda i: (i, 0)),
            pl.BlockSpec((BLOCK, K), lambda i: (i, 0)),
        ],
        out_shape=[
            jax.ShapeDtypeStruct((N_TOK, K), jnp.float32),
            jax.ShapeDtypeStruct((N_TOK, K), jnp.int32),
        ],
    )(hidden_states, weight)
    return (out_w, out_i)


# speedup vs baseline: 2.5018x; 1.0052x over previous
import jax
import jax.numpy as jnp
from jax.experimental import pallas as pl

N_TOK = 8192
N_EXP = 64
K = 8
BLOCK = 4096
NEG_INF = float("-inf")


def _gate_kernel(x_ref, w_ref, out_w_ref, out_i_ref):
    x = x_ref[...]
    w = w_ref[...]
    lt = jax.lax.dot_general(
        w, x, (((1,), (1,)), ((), ())), preferred_element_type=jnp.float32
    )
    eidx = jax.lax.broadcasted_iota(jnp.int32, lt.shape, 0)
    kiota = jax.lax.broadcasted_iota(jnp.int32, (K, BLOCK), 0)

    l = lt
    vals = jnp.zeros((K, BLOCK), jnp.float32)
    idxs = jnp.zeros((K, BLOCK), jnp.int32)
    m = None
    for k in range(K):
        cur = jnp.max(l, axis=0, keepdims=True)
        idx = jnp.min(jnp.where(l == cur, eidx, N_EXP), axis=0, keepdims=True)
        vals = jnp.where(kiota == k, cur, vals)
        idxs = jnp.where(kiota == k, idx, idxs)
        if k == 0:
            m = cur
            s = jnp.sum(jnp.exp(lt - m), axis=0, keepdims=True)
        if k + 1 < K:
            l = jnp.where(eidx == idx, NEG_INF, l)

    wts = jnp.exp(vals - m) / s
    out_w_ref[...] = wts.T
    out_i_ref[...] = idxs.T


@jax.jit
def kernel(hidden_states, weight):
    grid = (N_TOK // BLOCK,)
    out_w, out_i = pl.pallas_call(
        _gate_kernel,
        grid=grid,
        in_specs=[
            pl.BlockSpec((BLOCK, N_EXP), lambda i: (i, 0)),
            pl.BlockSpec((N_EXP, N_EXP), lambda i: (0, 0)),
        ],
        out_specs=[
            pl.BlockSpec((BLOCK, K), lambda i: (i, 0)),
            pl.BlockSpec((BLOCK, K), lambda i: (i, 0)),
        ],
        out_shape=[
            jax.ShapeDtypeStruct((N_TOK, K), jnp.float32),
            jax.ShapeDtypeStruct((N_TOK, K), jnp.int32),
        ],
    )(hidden_states, weight)
    return (out_w, out_i)
